# Initial kernel scaffold; baseline (speedup 1.0000x reference)
import functools

import jax
import jax.numpy as jnp
import numpy as np
from jax import lax
from jax.experimental import pallas as pl
from jax.experimental.pallas import tpu as pltpu
from jax.experimental.pallas import tpu_sc as plsc

N = 160000
K = 16
NC, NS = 2, 16
NW = NC * NS
ROWS_W = N // NW
CH = 1000
NCH = ROWS_W // CH
GB = 125
NGB = CH // GB


def _sc_gather(dist_hbm, angle_hbm, idxh_hbm, idxt_hbm,
               dist_out, angle_out, mm_out,
               idx_v, cidx_v, drows_v, arows_v, dout_v, aout_v, mm_v, sem):
    wid = lax.axis_index("s") * NC + lax.axis_index("c")
    base_w = wid * ROWS_W

    def chunk_body(ch, carry):
        mn, mx = carry
        base = base_w + ch * CH
        pltpu.sync_copy(idxh_hbm.at[pl.ds(base, CH)], idx_v)
        pltpu.sync_copy(idxt_hbm.at[pl.ds(base, CH), :], cidx_v)
        copies = []
        for b in range(NGB):
            sl = pl.ds(b * GB, GB)
            copies.append(pltpu.async_copy(
                dist_hbm.at[idx_v.at[sl]], drows_v.at[sl, :], sem))
            copies.append(pltpu.async_copy(
                angle_hbm.at[idx_v.at[sl]], arows_v.at[sl, :], sem))
        for c in copies:
            c.wait()

        def row_body(i, rc):
            rmn, rmx = rc
            colraw = cidx_v[i, :]
            col = jnp.minimum(colraw, K - 1)
            row = jnp.full((16,), i, jnp.int32)
            msk = colraw < K
            d = plsc.load_gather(drows_v, [row, col])
            a = plsc.load_gather(arows_v, [row, col])
            d = jnp.where(msk, d, 0.0)
            a = jnp.where(msk, a, 0.0)
            dout_v[i, :] = d
            aout_v[i, :] = a
            return jnp.minimum(rmn, d), jnp.maximum(rmx, d)

        mn, mx = lax.fori_loop(0, CH, row_body, (mn, mx))
        pltpu.sync_copy(dout_v, dist_out.at[pl.ds(base, CH), :])
        pltpu.sync_copy(aout_v, angle_out.at[pl.ds(base, CH), :])
        return mn, mx

    mn0 = jnp.full((16,), jnp.inf, jnp.float32)
    mx0 = jnp.full((16,), -jnp.inf, jnp.float32)
    mn, mx = lax.fori_loop(0, NCH, chunk_body, (mn0, mx0))
    mm_v[0, :] = mn
    mm_v[1, :] = mx
    pltpu.sync_copy(mm_v.at[0], mm_out.at[0, wid])
    pltpu.sync_copy(mm_v.at[1], mm_out.at[1, wid])


_sc_gather_call = functools.partial(
    pl.kernel,
    out_type=[
        jax.ShapeDtypeStruct((N, K), jnp.float32),
        jax.ShapeDtypeStruct((N, K), jnp.float32),
        jax.ShapeDtypeStruct((2, NW, 16), jnp.float32),
    ],
    mesh=plsc.VectorSubcoreMesh(core_axis_name="c", subcore_axis_name="s"),
    scratch_types=[
        pltpu.VMEM((CH,), jnp.int32),
        pltpu.VMEM((CH, K), jnp.int32),
        pltpu.VMEM((CH, K), jnp.float32),
        pltpu.VMEM((CH, K), jnp.float32),
        pltpu.VMEM((CH, K), jnp.float32),
        pltpu.VMEM((CH, K), jnp.float32),
        pltpu.VMEM((2, 16), jnp.float32),
        pltpu.SemaphoreType.DMA,
    ],
)(_sc_gather)


R = 256
OH = 352


def _tc_assemble(mm_ref, idx_ref, dist_ref, ang_ref, s_ref, mod_ref, out_ref):
    gmin = jnp.min(mm_ref[0])
    gmax = jnp.max(mm_ref[1])
    inv = 1.0 / (gmax - gmin)
    idx_f = idx_ref[...].astype(jnp.float32)
    rep = jnp.dot(idx_f, s_ref[...], preferred_element_type=jnp.float32)
    oh = jnp.where(rep == mod_ref[...], 1.0, 0.0)
    out_ref[:, 0:OH] = oh
    out_ref[:, OH:OH + K] = (dist_ref[...] - gmin) * inv
    out_ref[:, OH + K:OH + 2 * K] = ang_ref[...]


def kernel(dist, angle, idx_t, index_t, index_h, device):
    del device
    idx_t = idx_t.astype(jnp.int32)
    index_t = index_t.astype(jnp.int32)
    index_h = index_h.astype(jnp.int32)

    dist_t, angle_t, mm = _sc_gather_call(dist, angle, index_h, index_t)

    cols = np.arange(OH)
    s_mat = jnp.asarray((cols[None, :] // 22 == np.arange(K)[:, None])
                        .astype(np.float32))
    mod = jnp.asarray((cols % 22).astype(np.float32)[None, :])

    out = pl.pallas_call(
        _tc_assemble,
        grid=(N // R,),
        in_specs=[
            pl.BlockSpec((2, NW, 16), lambda i: (0, 0, 0)),
            pl.BlockSpec((R, K), lambda i: (i, 0)),
            pl.BlockSpec((R, K), lambda i: (i, 0)),
            pl.BlockSpec((R, K), lambda i: (i, 0)),
            pl.BlockSpec((K, OH), lambda i: (0, 0)),
            pl.BlockSpec((1, OH), lambda i: (0, 0)),
        ],
        out_specs=pl.BlockSpec((R, OH + 2 * K), lambda i: (i, 0)),
        out_shape=jax.ShapeDtypeStruct((N, OH + 2 * K), jnp.float32),
        compiler_params=pltpu.CompilerParams(
            dimension_semantics=("arbitrary",)),
    )(mm, idx_t, dist_t, angle_t, s_mat, mod)
    return out


# speedup vs baseline: 12.7069x; 12.7069x over previous
import functools

import jax
import jax.numpy as jnp
import numpy as np
from jax import lax
from jax.experimental import pallas as pl
from jax.experimental.pallas import tpu as pltpu
from jax.experimental.pallas import tpu_sc as plsc

N = 160000
K = 16
NC, NS = 2, 16
NW = NC * NS
ROWS_W = N // NW
CH = 1000
NCH = ROWS_W // CH
_GSLICES = [(o, min(128, CH - o)) for o in range(0, CH, 128)]


def _sc_gather(dist_hbm, angle_hbm, idxh_hbm, idxt_hbm,
               dist_out, angle_out, mm_out,
               idx_v, cidx_v, drows_v, arows_v, dout_v, aout_v, mm_v, sem):
    wid = lax.axis_index("s") * NC + lax.axis_index("c")
    base_w = wid * ROWS_W

    def chunk_body(ch, carry):
        mn, mx = carry
        base = base_w + ch * CH
        pltpu.sync_copy(idxh_hbm.at[pl.ds(base, CH)], idx_v)
        pltpu.sync_copy(idxt_hbm.at[pl.ds(base, CH), :], cidx_v)
        copies = []
        for off, sz in _GSLICES:
            sl = pl.ds(off, sz)
            copies.append(pltpu.async_copy(
                dist_hbm.at[idx_v.at[sl]], drows_v.at[sl, :], sem))
            copies.append(pltpu.async_copy(
                angle_hbm.at[idx_v.at[sl]], arows_v.at[sl, :], sem))
        for c in copies:
            c.wait()

        def row_body(i, rc):
            rmn, rmx = rc
            colraw = cidx_v[i, :]
            col = jnp.minimum(colraw, K - 1)
            row = jnp.full((16,), i, jnp.int32)
            msk = colraw < K
            d = plsc.load_gather(drows_v, [row, col])
            a = plsc.load_gather(arows_v, [row, col])
            d = jnp.where(msk, d, 0.0)
            a = jnp.where(msk, a, 0.0)
            dout_v[i, :] = d
            aout_v[i, :] = a
            return jnp.minimum(rmn, d), jnp.maximum(rmx, d)

        mn, mx = lax.fori_loop(0, CH, row_body, (mn, mx))
        pltpu.sync_copy(dout_v, dist_out.at[pl.ds(base, CH), :])
        pltpu.sync_copy(aout_v, angle_out.at[pl.ds(base, CH), :])
        return mn, mx

    mn0 = jnp.full((16,), jnp.inf, jnp.float32)
    mx0 = jnp.full((16,), -jnp.inf, jnp.float32)
    mn, mx = lax.fori_loop(0, NCH, chunk_body, (mn0, mx0))
    mm_v[0, :] = mn
    mm_v[1, :] = mx
    pltpu.sync_copy(mm_v.at[0], mm_out.at[0, wid])
    pltpu.sync_copy(mm_v.at[1], mm_out.at[1, wid])


@functools.lru_cache(maxsize=1)
def _sc_gather_call():
    return functools.partial(
        pl.kernel,
        out_type=[
            jax.ShapeDtypeStruct((N, K), jnp.float32),
            jax.ShapeDtypeStruct((N, K), jnp.float32),
            jax.ShapeDtypeStruct((2, NW, 16), jnp.float32),
        ],
        mesh=plsc.VectorSubcoreMesh(
            core_axis_name="c", subcore_axis_name="s",
            num_cores=NC, num_subcores=NS),
        compiler_params=pltpu.CompilerParams(
            needs_layout_passes=False, use_tc_tiling_on_sc=False),
        scratch_types=[
            pltpu.VMEM((CH,), jnp.int32),
            pltpu.VMEM((CH, K), jnp.int32),
            pltpu.VMEM((CH, K), jnp.float32),
            pltpu.VMEM((CH, K), jnp.float32),
            pltpu.VMEM((CH, K), jnp.float32),
            pltpu.VMEM((CH, K), jnp.float32),
            pltpu.VMEM((2, 16), jnp.float32),
            pltpu.SemaphoreType.DMA,
        ],
    )(_sc_gather)


R = 256
OH = 352


def _tc_assemble(mm_ref, idx_ref, dist_ref, ang_ref, s_ref, mod_ref, out_ref):
    gmin = jnp.min(mm_ref[0])
    gmax = jnp.max(mm_ref[1])
    inv = 1.0 / (gmax - gmin)
    idx_f = idx_ref[...].astype(jnp.float32)
    rep = jnp.dot(idx_f, s_ref[...], preferred_element_type=jnp.float32)
    oh = jnp.where(rep == mod_ref[...], 1.0, 0.0)
    out_ref[:, 0:OH] = oh
    out_ref[:, OH:OH + K] = (dist_ref[...] - gmin) * inv
    out_ref[:, OH + K:OH + 2 * K] = ang_ref[...]


def kernel(dist, angle, idx_t, index_t, index_h, device):
    del device
    idx_t = idx_t.astype(jnp.int32)
    index_t = index_t.astype(jnp.int32)
    index_h = index_h.astype(jnp.int32)

    dist_t, angle_t, mm = _sc_gather_call()(dist, angle, index_h, index_t)

    cols = np.arange(OH)
    s_mat = jnp.asarray((cols[None, :] // 22 == np.arange(K)[:, None])
                        .astype(np.float32))
    mod = jnp.asarray((cols % 22).astype(np.float32)[None, :])

    out = pl.pallas_call(
        _tc_assemble,
        grid=(N // R,),
        in_specs=[
            pl.BlockSpec((2, NW, 16), lambda i: (0, 0, 0)),
            pl.BlockSpec((R, K), lambda i: (i, 0)),
            pl.BlockSpec((R, K), lambda i: (i, 0)),
            pl.BlockSpec((R, K), lambda i: (i, 0)),
            pl.BlockSpec((K, OH), lambda i: (0, 0)),
            pl.BlockSpec((1, OH), lambda i: (0, 0)),
        ],
        out_specs=pl.BlockSpec((R, OH + 2 * K), lambda i: (i, 0)),
        out_shape=jax.ShapeDtypeStruct((N, OH + 2 * K), jnp.float32),
        compiler_params=pltpu.CompilerParams(
            dimension_semantics=("arbitrary",)),
    )(mm, idx_t, dist_t, angle_t, s_mat, mod)
    return out


# speedup vs baseline: 13.0024x; 1.0233x over previous
import functools

import jax
import jax.numpy as jnp
import numpy as np
from jax import lax
from jax.experimental import pallas as pl
from jax.experimental.pallas import tpu as pltpu
from jax.experimental.pallas import tpu_sc as plsc

N = 160000
K = 16
NC, NS = 2, 16
NW = NC * NS
ROWS_W = N // NW
CH = 1000
NCH = ROWS_W // CH
PC = 200
NPC = CH // PC
_GSLICES = [(o, min(128, CH - o)) for o in range(0, CH, 128)]


def _sc_gather(dist_hbm, angle_hbm, idxh_hbm, idxt_hbm, icls_hbm,
               combo_out, mm_out,
               idx_v, cidx_v, icls_v, drows_v, arows_v, co_v, mm_v, sem):
    wid = lax.axis_index("s") * NC + lax.axis_index("c")
    base_w = wid * ROWS_W

    def chunk_body(ch, carry):
        mn, mx = carry
        base = base_w + ch * CH
        pltpu.sync_copy(idxh_hbm.at[pl.ds(base, CH)], idx_v)
        pltpu.sync_copy(idxt_hbm.at[pl.ds(base, CH), :], cidx_v)
        pltpu.sync_copy(icls_hbm.at[pl.ds(base, CH), :], icls_v)
        copies = []
        for off, sz in _GSLICES:
            sl = pl.ds(off, sz)
            copies.append(pltpu.async_copy(
                dist_hbm.at[idx_v.at[sl]], drows_v.at[sl, :], sem))
            copies.append(pltpu.async_copy(
                angle_hbm.at[idx_v.at[sl]], arows_v.at[sl, :], sem))
        for c in copies:
            c.wait()

        for p in range(NPC):
            def row_body(i, rc, p=p):
                rmn, rmx = rc
                ic = p * PC + i
                colraw = cidx_v[ic, :]
                col = jnp.minimum(colraw, K - 1)
                row = jnp.full((16,), ic, jnp.int32)
                msk = colraw < K
                d = plsc.load_gather(drows_v, [row, col])
                a = plsc.load_gather(arows_v, [row, col])
                d = jnp.where(msk, d, 0.0)
                a = jnp.where(msk, a, 0.0)
                co_v[i, 0:K] = d
                co_v[i, K:2 * K] = a
                co_v[i, 2 * K:3 * K] = icls_v[ic, :].astype(jnp.float32)
                return jnp.minimum(rmn, d), jnp.maximum(rmx, d)

            mn, mx = lax.fori_loop(0, PC, row_body, (mn, mx))
            pltpu.sync_copy(co_v, combo_out.at[pl.ds(base + p * PC, PC), :])
        return mn, mx

    mn0 = jnp.full((16,), jnp.inf, jnp.float32)
    mx0 = jnp.full((16,), -jnp.inf, jnp.float32)
    mn, mx = lax.fori_loop(0, NCH, chunk_body, (mn0, mx0))
    mm_v[0, :] = mn
    mm_v[1, :] = mx
    pltpu.sync_copy(mm_v.at[0], mm_out.at[0, wid])
    pltpu.sync_copy(mm_v.at[1], mm_out.at[1, wid])


@functools.lru_cache(maxsize=1)
def _sc_gather_call():
    return functools.partial(
        pl.kernel,
        out_type=[
            jax.ShapeDtypeStruct((N, 128), jnp.float32),
            jax.ShapeDtypeStruct((2, NW, 16), jnp.float32),
        ],
        mesh=plsc.VectorSubcoreMesh(
            core_axis_name="c", subcore_axis_name="s",
            num_cores=NC, num_subcores=NS),
        compiler_params=pltpu.CompilerParams(
            needs_layout_passes=False, use_tc_tiling_on_sc=False),
        scratch_types=[
            pltpu.VMEM((CH,), jnp.int32),
            pltpu.VMEM((CH, K), jnp.int32),
            pltpu.VMEM((CH, K), jnp.int32),
            pltpu.VMEM((CH, K), jnp.float32),
            pltpu.VMEM((CH, K), jnp.float32),
            pltpu.VMEM((PC, 128), jnp.float32),
            pltpu.VMEM((2, 16), jnp.float32),
            pltpu.SemaphoreType.DMA,
        ],
    )(_sc_gather)


R = 256
OH = 352


def _tc_assemble(mm_ref, combo_ref, s_ref, mod_ref, out_ref):
    gmin = jnp.min(mm_ref[0])
    gmax = jnp.max(mm_ref[1])
    inv = 1.0 / (gmax - gmin)
    blk = combo_ref[...]
    dist = blk[:, 0:K]
    ang = blk[:, K:2 * K]
    idx_f = blk[:, 2 * K:3 * K]
    rep = jnp.dot(idx_f, s_ref[...], preferred_element_type=jnp.float32)
    oh = jnp.where(rep == mod_ref[...], 1.0, 0.0)
    out_ref[:, 0:OH] = oh
    out_ref[:, OH:OH + K] = (dist - gmin) * inv
    out_ref[:, OH + K:OH + 2 * K] = ang


def kernel(dist, angle, idx_t, index_t, index_h, device):
    del device
    idx_t = idx_t.astype(jnp.int32)
    index_t = index_t.astype(jnp.int32)
    index_h = index_h.astype(jnp.int32)

    combo, mm = _sc_gather_call()(dist, angle, index_h, index_t, idx_t)

    cols = np.arange(OH)
    s_mat = jnp.asarray((cols[None, :] // 22 == np.arange(K)[:, None])
                        .astype(np.float32))
    mod = jnp.asarray((cols % 22).astype(np.float32)[None, :])

    out = pl.pallas_call(
        _tc_assemble,
        grid=(N // R,),
        in_specs=[
            pl.BlockSpec((2, NW, 16), lambda i: (0, 0, 0)),
            pl.BlockSpec((R, 128), lambda i: (i, 0)),
            pl.BlockSpec((K, OH), lambda i: (0, 0)),
            pl.BlockSpec((1, OH), lambda i: (0, 0)),
        ],
        out_specs=pl.BlockSpec((R, OH + 2 * K), lambda i: (i, 0)),
        out_shape=jax.ShapeDtypeStruct((N, OH + 2 * K), jnp.float32),
        compiler_params=pltpu.CompilerParams(
            dimension_semantics=("arbitrary",)),
    )(mm, combo, s_mat, mod)
    return out


# speedup vs baseline: 17.2803x; 1.3290x over previous
import functools

import jax
import jax.numpy as jnp
import numpy as np
from jax import lax
from jax.experimental import pallas as pl
from jax.experimental.pallas import tpu as pltpu
from jax.experimental.pallas import tpu_sc as plsc

N = 160000
K = 16
NC, NS = 2, 16
NW = NC * NS
ROWS_W = N // NW
CH = 1000
NCH = ROWS_W // CH
PC = 200
NPC = CH // PC
_GSLICES = [(o, min(128, CH - o)) for o in range(0, CH, 128)]


def _sc_gather(dist_hbm, angle_hbm, idxh_hbm, idxt_hbm, icls_hbm,
               combo_out, mm_out,
               idx_v, cidx_v, icls_v, drows_v, arows_v, co_v, mm_v, sem):
    wid = lax.axis_index("s") * NC + lax.axis_index("c")
    base_w = wid * ROWS_W

    def chunk_body(ch, carry):
        mn, mx = carry
        base = base_w + ch * CH
        pltpu.sync_copy(idxh_hbm.at[pl.ds(base, CH)], idx_v)
        pltpu.sync_copy(idxt_hbm.at[pl.ds(base, CH), :], cidx_v)
        pltpu.sync_copy(icls_hbm.at[pl.ds(base, CH), :], icls_v)
        copies = []
        for off, sz in _GSLICES:
            sl = pl.ds(off, sz)
            copies.append(pltpu.async_copy(
                dist_hbm.at[idx_v.at[sl]], drows_v.at[sl, :], sem))
            copies.append(pltpu.async_copy(
                angle_hbm.at[idx_v.at[sl]], arows_v.at[sl, :], sem))
        for c in copies:
            c.wait()

        for p in range(NPC):
            def row_body(i, rc, p=p):
                rmn, rmx = rc
                ic = p * PC + i
                colraw = cidx_v[ic, :]
                col = jnp.minimum(colraw, K - 1)
                row = jnp.full((16,), ic, jnp.int32)
                msk = colraw < K
                d = plsc.load_gather(drows_v, [row, col])
                a = plsc.load_gather(arows_v, [row, col])
                d = jnp.where(msk, d, 0.0)
                a = jnp.where(msk, a, 0.0)
                co_v[i, 0:K] = icls_v[ic, :].astype(jnp.float32)
                co_v[i, K:2 * K] = d
                co_v[i, 2 * K:3 * K] = a
                return jnp.minimum(rmn, d), jnp.maximum(rmx, d)

            mn, mx = lax.fori_loop(0, PC, row_body, (mn, mx))
            pltpu.sync_copy(co_v, combo_out.at[pl.ds(base + p * PC, PC), :])
        return mn, mx

    mn0 = jnp.full((16,), jnp.inf, jnp.float32)
    mx0 = jnp.full((16,), -jnp.inf, jnp.float32)
    mn, mx = lax.fori_loop(0, NCH, chunk_body, (mn0, mx0))
    mm_v[0, :] = mn
    mm_v[1, :] = mx
    pltpu.sync_copy(mm_v.at[0], mm_out.at[0, wid])
    pltpu.sync_copy(mm_v.at[1], mm_out.at[1, wid])


@functools.lru_cache(maxsize=1)
def _sc_gather_call():
    return functools.partial(
        pl.kernel,
        out_type=[
            jax.ShapeDtypeStruct((N, 128), jnp.float32),
            jax.ShapeDtypeStruct((2, NW, 16), jnp.float32),
        ],
        mesh=plsc.VectorSubcoreMesh(
            core_axis_name="c", subcore_axis_name="s",
            num_cores=NC, num_subcores=NS),
        compiler_params=pltpu.CompilerParams(
            needs_layout_passes=False, use_tc_tiling_on_sc=False),
        scratch_types=[
            pltpu.VMEM((CH,), jnp.int32),
            pltpu.VMEM((CH, K), jnp.int32),
            pltpu.VMEM((CH, K), jnp.int32),
            pltpu.VMEM((CH, K), jnp.float32),
            pltpu.VMEM((CH, K), jnp.float32),
            pltpu.VMEM((PC, 128), jnp.float32),
            pltpu.VMEM((2, 16), jnp.float32),
            pltpu.SemaphoreType.DMA,
        ],
    )(_sc_gather)


R = 512
OH = 352


def _tc_assemble(mm_ref, combo_ref, s_ref, mod_ref, out_ref):
    gmin = jnp.min(mm_ref[0])
    gmax = jnp.max(mm_ref[1])
    inv = 1.0 / (gmax - gmin)
    blk = combo_ref[...]
    idx_f = blk[:, 0:K]
    dist = blk[:, K:2 * K]
    ang = blk[:, 2 * K:3 * K]
    rep = jnp.dot(idx_f, s_ref[...], preferred_element_type=jnp.float32)
    oh = jnp.where(rep == mod_ref[...], 1.0, 0.0)
    out_ref[:, 0:OH] = oh
    out_ref[:, OH:OH + K] = (dist - gmin) * inv
    out_ref[:, OH + K:OH + 2 * K] = ang


def kernel(dist, angle, idx_t, index_t, index_h, device):
    del device
    idx_t = idx_t.astype(jnp.int32)
    index_t = index_t.astype(jnp.int32)
    index_h = index_h.astype(jnp.int32)

    combo, mm = _sc_gather_call()(dist, angle, index_h, index_t, idx_t)

    cols = np.arange(OH)
    s_mat = jnp.asarray((cols[None, :] // 22 == np.arange(K)[:, None])
                        .astype(np.float32))
    mod = jnp.asarray((cols % 22).astype(np.float32)[None, :])

    out = pl.pallas_call(
        _tc_assemble,
        grid=(N // R,),
        in_specs=[
            pl.BlockSpec((2, NW, 16), lambda i: (0, 0, 0)),
            pl.BlockSpec((R, 128), lambda i: (i, 0)),
            pl.BlockSpec((K, OH), lambda i: (0, 0)),
            pl.BlockSpec((1, OH), lambda i: (0, 0)),
        ],
        out_specs=pl.BlockSpec((R, OH + 2 * K), lambda i: (i, 0)),
        out_shape=jax.ShapeDtypeStruct((N, OH + 2 * K), jnp.float32),
        compiler_params=pltpu.CompilerParams(
            dimension_semantics=("arbitrary",)),
    )(mm, combo, s_mat, mod)
    return out


# speedup vs baseline: 18.2812x; 1.0579x over previous
import functools

import jax
import jax.numpy as jnp
import numpy as np
from jax import lax
from jax.experimental import pallas as pl
from jax.experimental.pallas import tpu as pltpu
from jax.experimental.pallas import tpu_sc as plsc

N = 160000
K = 16
NC, NS = 2, 16
NW = NC * NS
ROWS_W = N // NW
CH = 1000
NCH = ROWS_W // CH
PC = 200
NPC = CH // PC
_GSLICES = [(o, min(128, CH - o)) for o in range(0, CH, 128)]


def _sc_gather(dist_hbm, angle_hbm, idxh_hbm, idxt_hbm, icls_hbm,
               combo_out, mm_out,
               idx_v, cidx_v, icls_v, drows_v, arows_v, co_v, mm_v, sem):
    wid = lax.axis_index("s") * NC + lax.axis_index("c")
    base_w = wid * ROWS_W

    def chunk_body(ch, carry):
        mn, mx = carry
        base = base_w + ch * CH
        pltpu.sync_copy(idxh_hbm.at[pl.ds(base, CH)], idx_v)
        pltpu.sync_copy(idxt_hbm.at[pl.ds(base, CH), :], cidx_v)
        pltpu.sync_copy(icls_hbm.at[pl.ds(base, CH), :], icls_v)
        copies = []
        for off, sz in _GSLICES:
            sl = pl.ds(off, sz)
            copies.append(pltpu.async_copy(
                dist_hbm.at[idx_v.at[sl]], drows_v.at[sl, :], sem))
            copies.append(pltpu.async_copy(
                angle_hbm.at[idx_v.at[sl]], arows_v.at[sl, :], sem))
        for c in copies:
            c.wait()

        for p in range(NPC):
            def row_body(i, rc, p=p):
                rmn, rmx = rc
                ic = p * PC + i
                colraw = cidx_v[ic, :]
                col = jnp.minimum(colraw, K - 1)
                row = jnp.full((16,), ic, jnp.int32)
                msk = colraw < K
                d = plsc.load_gather(drows_v, [row, col])
                a = plsc.load_gather(arows_v, [row, col])
                d = jnp.where(msk, d, 0.0)
                a = jnp.where(msk, a, 0.0)
                co_v[i, 0:K] = icls_v[ic, :].astype(jnp.float32)
                co_v[i, K:2 * K] = d
                co_v[i, 2 * K:3 * K] = a
                return jnp.minimum(rmn, d), jnp.maximum(rmx, d)

            mn, mx = lax.fori_loop(0, PC, row_body, (mn, mx))
            pltpu.sync_copy(co_v, combo_out.at[pl.ds(base + p * PC, PC), :])
        return mn, mx

    mn0 = jnp.full((16,), jnp.inf, jnp.float32)
    mx0 = jnp.full((16,), -jnp.inf, jnp.float32)
    mn, mx = lax.fori_loop(0, NCH, chunk_body, (mn0, mx0))
    mm_v[0, :] = mn
    mm_v[1, :] = mx
    pltpu.sync_copy(mm_v.at[0], mm_out.at[0, wid])
    pltpu.sync_copy(mm_v.at[1], mm_out.at[1, wid])


@functools.lru_cache(maxsize=1)
def _sc_gather_call():
    return functools.partial(
        pl.kernel,
        out_type=[
            jax.ShapeDtypeStruct((N, 128), jnp.float32),
            jax.ShapeDtypeStruct((2, NW, 16), jnp.float32),
        ],
        mesh=plsc.VectorSubcoreMesh(
            core_axis_name="c", subcore_axis_name="s",
            num_cores=NC, num_subcores=NS),
        compiler_params=pltpu.CompilerParams(
            needs_layout_passes=False, use_tc_tiling_on_sc=False),
        scratch_types=[
            pltpu.VMEM((CH,), jnp.int32),
            pltpu.VMEM((CH, K), jnp.int32),
            pltpu.VMEM((CH, K), jnp.int32),
            pltpu.VMEM((CH, K), jnp.float32),
            pltpu.VMEM((CH, K), jnp.float32),
            pltpu.VMEM((PC, 128), jnp.float32),
            pltpu.VMEM((2, 16), jnp.float32),
            pltpu.SemaphoreType.DMA,
        ],
    )(_sc_gather)


R = 640
OH = 352


def _tc_assemble(mm_ref, combo_ref, s_ref, mod_ref, out_ref):
    gmin = jnp.min(mm_ref[0])
    gmax = jnp.max(mm_ref[1])
    inv = 1.0 / (gmax - gmin)
    blk = combo_ref[...]
    idx_f = blk[:, 0:K]
    dist = blk[:, K:2 * K]
    ang = blk[:, 2 * K:3 * K]
    rep = jnp.dot(idx_f, s_ref[...], preferred_element_type=jnp.float32)
    oh = jnp.where(rep == mod_ref[...], 1.0, 0.0)
    out_ref[:, 0:OH] = oh
    out_ref[:, OH:OH + K] = (dist - gmin) * inv
    out_ref[:, OH + K:OH + 2 * K] = ang


def kernel(dist, angle, idx_t, index_t, index_h, device):
    del device
    idx_t = idx_t.astype(jnp.int32)
    index_t = index_t.astype(jnp.int32)
    index_h = index_h.astype(jnp.int32)

    combo, mm = _sc_gather_call()(dist, angle, index_h, index_t, idx_t)

    cols = np.arange(OH)
    s_mat = jnp.asarray((cols[None, :] // 22 == np.arange(K)[:, None])
                        .astype(np.float32))
    mod = jnp.asarray((cols % 22).astype(np.float32)[None, :])

    out = pl.pallas_call(
        _tc_assemble,
        grid=(N // R,),
        in_specs=[
            pl.BlockSpec((2, NW, 16), lambda i: (0, 0, 0)),
            pl.BlockSpec((R, 128), lambda i: (i, 0)),
            pl.BlockSpec((K, OH), lambda i: (0, 0)),
            pl.BlockSpec((1, OH), lambda i: (0, 0)),
        ],
        out_specs=pl.BlockSpec((R, OH + 2 * K), lambda i: (i, 0)),
        out_shape=jax.ShapeDtypeStruct((N, OH + 2 * K), jnp.float32),
        compiler_params=pltpu.CompilerParams(
            dimension_semantics=("arbitrary",)),
    )(mm, combo, s_mat, mod)
    return out
